# baseline (speedup 1.0000x reference)
import functools

import jax
import jax.numpy as jnp
from jax import lax
from jax.experimental import pallas as pl
from jax.experimental.pallas import tpu as pltpu
from jax.experimental.pallas import tpu_sc as plsc

NUM_ROWS = 1000000
DIM = 64
BATCH = 4096
SEQ = 200
TOTAL = BATCH * SEQ

NC, NS = 2, 16
NW = NC * NS
PER_W = TOTAL // NW
CHUNK = 128
NCHUNK = PER_W // CHUNK


def _sc_gather(idx_flat, w):
    mesh = plsc.VectorSubcoreMesh(core_axis_name="c", subcore_axis_name="s")

    @functools.partial(
        pl.kernel,
        out_type=jax.ShapeDtypeStruct((TOTAL, DIM), jnp.float32),
        mesh=mesh,
        scratch_types=[
            pltpu.VMEM((NCHUNK, CHUNK), jnp.int32),
            pltpu.VMEM((CHUNK, DIM), jnp.float32),
            pltpu.SemaphoreType.DMA,
        ],
        compiler_params=pltpu.CompilerParams(use_tc_tiling_on_sc=False),
    )
    def k(idx_hbm, w_hbm, out_hbm, idx_v, buf, sem):
        wid = lax.axis_index("s") * NC + lax.axis_index("c")
        base = wid * PER_W
        pltpu.sync_copy(idx_hbm.at[wid], idx_v)

        def body(j, carry):
            pltpu.async_copy(w_hbm.at[idx_v.at[j]], buf, sem).wait()
            pltpu.sync_copy(buf, out_hbm.at[pl.ds(base + j * CHUNK, CHUNK)])
            return carry

        lax.fori_loop(0, NCHUNK, body, 0)

    return k(idx_flat, w)


def kernel(token_ids, W):
    idx = token_ids.astype(jnp.int32).reshape(NW, NCHUNK, CHUNK)
    out = _sc_gather(idx, W)
    return out.reshape(BATCH, SEQ, DIM)


# speedup vs baseline: 1.1188x; 1.1188x over previous
import functools

import jax
import jax.numpy as jnp
from jax import lax
from jax.experimental import pallas as pl
from jax.experimental.pallas import tpu as pltpu
from jax.experimental.pallas import tpu_sc as plsc

NUM_ROWS = 1000000
DIM = 64
BATCH = 4096
SEQ = 200
TOTAL = BATCH * SEQ

NC, NS = 2, 16
NW = NC * NS
PER_W = TOTAL // NW
CHUNK = 128
NCHUNK = PER_W // CHUNK
NBUF = 8
K = 4


def _sc_gather(idx_flat, w):
    mesh = plsc.VectorSubcoreMesh(core_axis_name="c", subcore_axis_name="s")

    @functools.partial(
        pl.kernel,
        out_type=jax.ShapeDtypeStruct((TOTAL, DIM), jnp.float32),
        mesh=mesh,
        scratch_types=[
            pltpu.VMEM((NCHUNK, CHUNK), jnp.int32),
            pltpu.VMEM((NBUF, CHUNK, DIM), jnp.float32),
            [pltpu.SemaphoreType.DMA] * NBUF,
            [pltpu.SemaphoreType.DMA] * NBUF,
        ],
        compiler_params=pltpu.CompilerParams(use_tc_tiling_on_sc=False),
    )
    def k(idx_hbm, w_hbm, out_hbm, idx_v, buf, gsem, wsem):
        wid = lax.axis_index("s") * NC + lax.axis_index("c")
        base = wid * PER_W
        pltpu.sync_copy(idx_hbm.at[wid], idx_v)

        def gather_start(j, b):
            pltpu.async_copy(w_hbm.at[idx_v.at[j]], buf.at[b], gsem[b])

        def gather_wait(b):
            pltpu.make_async_copy(
                w_hbm.at[pl.ds(0, CHUNK)], buf.at[b], gsem[b]
            ).wait()

        def write_start(j, b):
            pltpu.async_copy(
                buf.at[b], out_hbm.at[pl.ds(base + j * CHUNK, CHUNK)], wsem[b]
            )

        def write_wait(b):
            pltpu.make_async_copy(
                buf.at[b], out_hbm.at[pl.ds(base, CHUNK)], wsem[b]
            ).wait()

        for b in range(K):
            gather_start(b, b)
        for b in range(K, NBUF):
            gather_start(b, b)
            gather_wait(b - K)
            write_start(b - K, b - K)

        def outer(g, carry):
            for bb in range(NBUF):
                j = g * NBUF + bb
                bw = (bb + K) % NBUF
                write_wait(bb)
                gather_start(j, bb)
                gather_wait(bw)
                write_start(j - K, bw)
            return carry

        lax.fori_loop(1, NCHUNK // NBUF, outer, 0)

        for jw in range(NCHUNK - K, NCHUNK):
            b = jw % NBUF
            gather_wait(b)
            write_start(jw, b)
        for b in range(NBUF):
            write_wait(b)

    return k(idx_flat, w)


def kernel(token_ids, W):
    idx = token_ids.astype(jnp.int32).reshape(NW, NCHUNK, CHUNK)
    out = _sc_gather(idx, W)
    return out.reshape(BATCH, SEQ, DIM)
